# baseline (speedup 1.0000x reference)
import functools

import jax
import jax.numpy as jnp
from jax import lax
from jax.experimental import pallas as pl
from jax.experimental.pallas import tpu as pltpu
from jax.experimental.pallas import tpu_sc as plsc

CHUNK = 128


@functools.lru_cache(maxsize=None)
def _make_sc_kernel(V, B, F):
    info = plsc.get_sparse_core_info()
    NC, NS, L = info.num_cores, info.num_subcores, info.num_lanes
    NW = NC * NS
    n_per = B // NW
    n_chunks = n_per // CHUNK
    groups = CHUNK // L
    NL = F // L

    mesh = plsc.VectorSubcoreMesh(core_axis_name="c", subcore_axis_name="s")

    @functools.partial(
        pl.kernel,
        out_type=jax.ShapeDtypeStruct((B,), jnp.float32),
        mesh=mesh,
        compiler_params=pltpu.CompilerParams(needs_layout_passes=False,
                                             use_tc_tiling_on_sc=False),
        scratch_types=[
            pltpu.VMEM((n_chunks, CHUNK), jnp.int32),
            pltpu.VMEM((n_chunks, CHUNK), jnp.int32),
            pltpu.VMEM((2, CHUNK, F), jnp.float32),
            pltpu.VMEM((2, CHUNK, F), jnp.float32),
            pltpu.VMEM((n_per,), jnp.float32),
            pltpu.VMEM((L * L,), jnp.float32),
            pltpu.VMEM((F,), jnp.float32),
            pltpu.VMEM((F,), jnp.float32),
            pltpu.SemaphoreType.DMA,
            pltpu.SemaphoreType.DMA,
        ],
    )
    def k(x_hbm, idsu_hbm, idsi_hbm, w1_hbm, w2_hbm, out_hbm,
          idx_u, idx_i, rows_u, rows_i, out_v, prod_v, wv1, wv2, sem0, sem1):
        wid = lax.axis_index("s") * NC + lax.axis_index("c")
        pltpu.sync_copy(idsu_hbm.at[pl.ds(wid * n_chunks, n_chunks)], idx_u)
        pltpu.sync_copy(idsi_hbm.at[pl.ds(wid * n_chunks, n_chunks)], idx_i)
        pltpu.sync_copy(w1_hbm, wv1)
        pltpu.sync_copy(w2_hbm, wv2)

        sems = (sem0, sem1)

        def fire(c):
            buf = c % 2
            du = pltpu.async_copy(x_hbm.at[idx_u.at[c]], rows_u.at[buf],
                                  sems[buf])
            di = pltpu.async_copy(x_hbm.at[idx_i.at[c]], rows_i.at[buf],
                                  sems[buf])
            return du, di

        w1l = [wv1[pl.ds(l * L, L)] for l in range(NL)]
        w2l = [wv2[pl.ds(l * L, L)] for l in range(NL)]

        def compute(c, buf):
            ru = rows_u.at[buf]
            ri = rows_i.at[buf]

            lane = lax.iota(jnp.int32, L)

            def body(g, carry):
                acc = jnp.zeros((L,), jnp.float32)
                for j in range(L):
                    e = g * L + j
                    uw = jnp.zeros((L,), jnp.float32)
                    iw = jnp.zeros((L,), jnp.float32)
                    for l in range(NL):
                        uw = uw + ru[e, pl.ds(l * L, L)] * w1l[l]
                        iw = iw + ri[e, pl.ds(l * L, L)] * w2l[l]
                    s = jnp.sum(uw * iw)
                    acc = jnp.where(lane == j, s, acc)
                out_v[pl.ds(c * CHUNK + g * L, L)] = acc
                return carry

            lax.fori_loop(0, groups, body, 0)

        descs = fire(0)
        for c in range(n_chunks):
            nxt = fire(c + 1) if c + 1 < n_chunks else None
            for d in descs:
                d.wait()
            compute(c, c % 2)
            descs = nxt

        pltpu.sync_copy(out_v, out_hbm.at[pl.ds(wid * n_per, n_per)])

    return k


def kernel(X, ids, w1, w2):
    V, NL, D = X.shape
    B = ids.shape[0]
    F = NL * D
    Xf = X.reshape(V, F)
    ids_u = ids[:, 0].reshape(-1, CHUNK)
    ids_i = ids[:, 1].reshape(-1, CHUNK)
    w1b = jnp.repeat(w1, D)
    w2b = jnp.repeat(w2, D)
    return _make_sc_kernel(V, B, F)(Xf, ids_u, ids_i, w1b, w2b)
